# baseline (speedup 1.0000x reference)
import functools
import jax
import jax.numpy as jnp
from jax import lax
from jax.experimental import pallas as pl
from jax.experimental.pallas import tpu as pltpu
from jax.experimental.pallas import tpu_sc as plsc

N = 10000
E = 320000
DIN = 128
H = 128
DE = 16

NC = 2
NS = 16
NW = NC * NS
CH = 40
ROWS = E // CH
_RSQRT_H = 1.0 / (128.0 ** 0.5)


def _proj_body(x_ref, w_ref, b_ref, t_ref, skip_ref):
    p = (
        jnp.dot(x_ref[...], w_ref[...], preferred_element_type=jnp.float32)
        + b_ref[...]
    )
    t_ref[0] = p[:, 0:H]
    t_ref[1] = p[:, H:2 * H]
    t_ref[2] = p[:, 2 * H:3 * H]
    skip_ref[...] = p[:, 3 * H:4 * H]


def _proj(x, w, b, bm):
    n = x.shape[0]
    return pl.pallas_call(
        _proj_body,
        grid=(n // bm,),
        in_specs=[
            pl.BlockSpec((bm, DIN), lambda i: (i, 0)),
            pl.BlockSpec((DIN, 4 * H), lambda i: (0, 0)),
            pl.BlockSpec((1, 4 * H), lambda i: (0, 0)),
        ],
        out_specs=[
            pl.BlockSpec((3, bm, H), lambda i: (0, i, 0)),
            pl.BlockSpec((bm, H), lambda i: (i, 0)),
        ],
        out_shape=[
            jax.ShapeDtypeStruct((3, n, H), jnp.float32),
            jax.ShapeDtypeStruct((n, H), jnp.float32),
        ],
    )(x, w, b.reshape(1, 4 * H))


def _norm_proj_body(a_ref, den_ref, skip_ref, w_ref, b_ref, o_ref, h_ref):
    num = a_ref[...]
    den = den_ref[...]
    h = jnp.maximum(num / (den + 1e-16) + skip_ref[...], 0.0)
    h_ref[...] = h
    o = jnp.dot(h, w_ref[...], preferred_element_type=jnp.float32) + b_ref[...]
    o_ref[0] = o[:, 0:H]
    o_ref[1] = o[:, H:2 * H]


def _norm_proj(slabs, den, skip, w, b, bm):
    n = skip.shape[0]
    dout = w.shape[1]
    return pl.pallas_call(
        _norm_proj_body,
        grid=(n // bm,),
        in_specs=[
            pl.BlockSpec((bm, H), lambda i: (i, 0)),
            pl.BlockSpec((bm, 1), lambda i: (i, 0)),
            pl.BlockSpec((bm, H), lambda i: (i, 0)),
            pl.BlockSpec((H, dout), lambda i: (0, 0)),
            pl.BlockSpec((1, dout), lambda i: (0, 0)),
        ],
        out_specs=[
            pl.BlockSpec((2, bm, H), lambda i: (0, i, 0)),
            pl.BlockSpec((bm, H), lambda i: (i, 0)),
        ],
        out_shape=[
            jax.ShapeDtypeStruct((2, n, H), jnp.float32),
            jax.ShapeDtypeStruct((n, H), jnp.float32),
        ],
    )(slabs, den, skip, w, b.reshape(1, dout))


def _fin_body(s_ref, den_ref, xl_ref, xr_ref, att_ref, bg_ref, o_ref):
    xl = xl_ref[...]
    s = xl + xr_ref[...]
    s = jnp.where(s > 0, s, 0.2 * s)
    eself = jnp.exp(jnp.sum(s * att_ref[...], axis=-1, keepdims=True))
    num = s_ref[...] + eself * xl
    den = den_ref[...] + eself + 1e-16
    o_ref[...] = num / den + bg_ref[...]


def _finalize(slabs, den, xl, xr, att, bg, bm):
    n = xl.shape[0]
    return pl.pallas_call(
        _fin_body,
        grid=(n // bm,),
        in_specs=[
            pl.BlockSpec((bm, H), lambda i: (i, 0)),
            pl.BlockSpec((bm, 1), lambda i: (i, 0)),
            pl.BlockSpec((bm, H), lambda i: (i, 0)),
            pl.BlockSpec((bm, H), lambda i: (i, 0)),
            pl.BlockSpec((1, H), lambda i: (0, 0)),
            pl.BlockSpec((1, H), lambda i: (0, 0)),
        ],
        out_specs=pl.BlockSpec((bm, H), lambda i: (i, 0)),
        out_shape=jax.ShapeDtypeStruct((n, H), jnp.float32),
    )(slabs, den, xl, xr, att.reshape(1, H), bg.reshape(1, H))


def _mlp_body(ea_ref, w1_ref, b1_ref, g_ref, be_ref, w2_ref, b2_ref, o_ref):
    he = (
        jnp.dot(ea_ref[...], w1_ref[...], preferred_element_type=jnp.float32)
        + b1_ref[...]
    )
    mu = jnp.mean(he, axis=-1, keepdims=True)
    var = jnp.mean((he - mu) ** 2, axis=-1, keepdims=True)
    he = (he - mu) * jax.lax.rsqrt(var + 1e-5) * g_ref[...] + be_ref[...]
    he = jnp.maximum(he, 0.0)
    o_ref[...] = (
        jnp.dot(he, w2_ref[...], preferred_element_type=jnp.float32)
        + b2_ref[...]
    )


def _edge_mlp(ea, w1, b1, gamma, beta, w2, b2, bm):
    e = ea.shape[0]
    return pl.pallas_call(
        _mlp_body,
        grid=(e // bm,),
        in_specs=[
            pl.BlockSpec((bm, DE), lambda i: (i, 0)),
            pl.BlockSpec((DE, H), lambda i: (0, 0)),
            pl.BlockSpec((1, H), lambda i: (0, 0)),
            pl.BlockSpec((1, H), lambda i: (0, 0)),
            pl.BlockSpec((1, H), lambda i: (0, 0)),
            pl.BlockSpec((H, H), lambda i: (0, 0)),
            pl.BlockSpec((1, H), lambda i: (0, 0)),
        ],
        out_specs=pl.BlockSpec((bm, H), lambda i: (i, 0)),
        out_shape=jax.ShapeDtypeStruct((e, H), jnp.float32),
    )(ea, w1, b1.reshape(1, H), gamma.reshape(1, H), beta.reshape(1, H),
      w2, b2.reshape(1, H))


_N_STRIPE = 624
_N_REM = N - NS * _N_STRIPE
DR = 80
N2 = N + DR


def _mk_sc_attn(mode):
    mesh = plsc.VectorSubcoreMesh(
        core_axis_name="c", subcore_axis_name="s", num_cores=NC,
        num_subcores=NS,
    )

    nbuf = 2
    idx_scr = [pltpu.VMEM((2, CH), jnp.int32) for _ in range(nbuf)]
    aidx_scr = [pltpu.VMEM((CH,), jnp.int32) for _ in range(nbuf)]
    vidx_scr = [pltpu.VMEM((CH,), jnp.int32) for _ in range(nbuf)]
    bidx_scr = [pltpu.VMEM((CH,), jnp.int32) for _ in range(nbuf)]
    scidx_scr = [pltpu.VMEM((2 * CH,), jnp.int32) for _ in range(nbuf)]
    a_scr = [pltpu.VMEM((CH, H), jnp.float32) for _ in range(nbuf)]
    b_scr = [pltpu.VMEM((CH, H), jnp.float32) for _ in range(nbuf)]
    o_scr = [pltpu.VMEM((2 * CH, H), jnp.float32) for _ in range(nbuf)]
    sems = [pltpu.SemaphoreType.DMA for _ in range(4 * nbuf)]

    @functools.partial(
        pl.kernel,
        out_type=jax.ShapeDtypeStruct((NC, N2, H), jnp.float32),
        mesh=mesh,
        scratch_types=(idx_scr + aidx_scr + vidx_scr + bidx_scr + scidx_scr
                       + a_scr + b_scr + o_scr
                       + [pltpu.VMEM((H,), jnp.float32)] + sems
                       + [pltpu.VMEM_SHARED((N2, H), jnp.float32)]),
    )
    def sc_attn(edges, tbl, attv, out, *scr):
        idxb = scr[0:2]
        aidx = scr[2:4]
        vidx = scr[4:6]
        bidx = scr[6:8]
        scidx = scr[8:10]
        abuf = scr[10:12]
        bbuf = scr[12:14]
        obuf = scr[14:16]
        attbuf = scr[16]
        sem_a = scr[17:19]
        sem_b = scr[19:21]
        sem_v = scr[21:23]
        sem_s = scr[23:25]
        slab = scr[25]

        c = lax.axis_index("c")
        s = lax.axis_index("s")
        w = s * NC + c
        lanes = lax.iota(jnp.int32, 16)

        def zrow(i, _):
            for j in range(8):
                obuf[0][i, pl.ds(16 * j, 16)] = jnp.zeros((16,), jnp.float32)
            return 0
        lax.fori_loop(0, 2 * CH, zrow, 0)

        nz = _N_STRIPE // (2 * CH)
        for t in range(nz):
            pltpu.sync_copy(
                obuf[0], slab.at[pl.ds(s * _N_STRIPE + t * 2 * CH, 2 * CH)])
        _rem = _N_STRIPE % (2 * CH)
        if _rem:
            pltpu.sync_copy(
                obuf[0].at[pl.ds(0, _rem)],
                slab.at[pl.ds(s * _N_STRIPE + _N_STRIPE - _rem, _rem)])

        @pl.when(s < (N2 - NS * _N_STRIPE) // 8)
        def _():
            pltpu.sync_copy(obuf[0].at[pl.ds(0, 8)],
                            slab.at[pl.ds(NS * _N_STRIPE + 8 * s, 8)])

        pltpu.sync_copy(attv, attbuf)
        plsc.subcore_barrier()

        attregs = [attbuf[pl.ds(16 * j, 16)] for j in range(8)]

        _gdn = lax.GatherDimensionNumbers(
            offset_dims=(), collapsed_slice_dims=(0,), start_index_map=(0,))

        def _shuf(vec, idx):
            return lax.gather(
                vec, idx[:, None], _gdn, (1,),
                mode=lax.GatherScatterMode.PROMISE_IN_BOUNDS)

        def _allsum(vec):
            for kk in (8, 4, 2, 1):
                vec = vec + _shuf(vec, lanes ^ kk)
            return vec

        nch = ROWS // NW
        base = w * nch

        gstarts = []
        g0 = 0
        while g0 + 16 <= CH:
            gstarts.append(g0)
            g0 += 16
        if gstarts[-1] + 16 < CH:
            gstarts.append(CH - 16)

        def stage(i, k):
            r = base + i
            pltpu.sync_copy(edges.at[r], idxb[k])
            for g in gstarts:
                sv = idxb[k][0, pl.ds(g, 16)]
                dv = idxb[k][1, pl.ds(g, 16)]
                if mode == "l1":
                    aidx[k][pl.ds(g, 16)] = sv + N
                    vidx[k][pl.ds(g, 16)] = sv + 2 * N
                    bidx[k][pl.ds(g, 16)] = dv
                else:
                    aidx[k][pl.ds(g, 16)] = sv
                    bidx[k][pl.ds(g, 16)] = dv + N
                scidx[k][pl.ds(g, 16)] = dv
                scidx[k][pl.ds(CH + g, 16)] = (
                    lax.shift_right_logical(dv, 7) + N)
            pltpu.async_copy(tbl.at[aidx[k]], abuf[k], sem_a[k])
            pltpu.async_copy(tbl.at[bidx[k]], bbuf[k], sem_b[k])
            if mode == "l1":
                pltpu.async_copy(tbl.at[vidx[k]],
                                 obuf[k].at[pl.ds(0, CH)], sem_v[k])

        def wait_gathers(k):
            pltpu.make_async_copy(tbl.at[aidx[k]], abuf[k], sem_a[k]).wait()
            pltpu.make_async_copy(tbl.at[bidx[k]], bbuf[k], sem_b[k]).wait()
            if mode == "l1":
                pltpu.make_async_copy(
                    tbl.at[vidx[k]], obuf[k].at[pl.ds(0, CH)],
                    sem_v[k]).wait()

        def wait_scatter(k):
            pltpu.make_async_copy(
                obuf[k], slab.at[scidx[k]], sem_s[k]).wait()

        def compute(k):
            @plsc.parallel_loop(0, CH, unroll=2)
            def _edge(e):
                if mode == "l1":
                    acc = jnp.zeros((16,), jnp.float32)
                    for j in range(8):
                        a = abuf[k][e, pl.ds(16 * j, 16)]
                        b = bbuf[k][e, pl.ds(16 * j, 16)]
                        acc = acc + a * b
                    ex = jnp.exp(_allsum(acc) * _RSQRT_H)
                    for j in range(8):
                        v = obuf[k][e, pl.ds(16 * j, 16)]
                        obuf[k][e, pl.ds(16 * j, 16)] = v * ex
                else:
                    acc = jnp.zeros((16,), jnp.float32)
                    pay = []
                    for j in range(8):
                        a = abuf[k][e, pl.ds(16 * j, 16)]
                        pay.append(a)
                        t = a + bbuf[k][e, pl.ds(16 * j, 16)]
                        t = jnp.where(t > 0, t, 0.2 * t)
                        acc = acc + t * attregs[j]
                    ex = jnp.exp(_allsum(acc))
                    for j in range(8):
                        obuf[k][e, pl.ds(16 * j, 16)] = pay[j] * ex
                gs = jnp.minimum((e // 16) * 16, CH - 16)
                dvec = idxb[k][1, pl.ds(gs, 16)]
                dlo = _shuf(dvec, jnp.full((16,), 0, jnp.int32) + (e - gs))
                dlo = dlo & 127
                for j in range(8):
                    obuf[k][CH + e, pl.ds(16 * j, 16)] = jnp.where(
                        lanes + 16 * j == dlo, ex, 0.0)

        stage(0, 0)

        def do_pair(i2, _):
            for b in range(2):
                i = 2 * i2 + b
                nb = 1 - b

                wait_gathers(b)
                compute(b)

                @pl.when(i + 1 < nch)
                def _():
                    @pl.when(i >= 1)
                    def _():
                        wait_scatter(nb)
                    stage(i + 1, nb)

                pltpu.async_copy(obuf[b], slab.at[scidx[b]], sem_s[b])
            return 0

        lax.fori_loop(0, nch // 2, do_pair, 0)
        wait_scatter(0)
        wait_scatter(1)

        plsc.subcore_barrier()
        pltpu.sync_copy(slab.at[pl.ds(s * _N_STRIPE, _N_STRIPE)],
                        out.at[c].at[pl.ds(s * _N_STRIPE, _N_STRIPE)])

        @pl.when(s < (N2 - NS * _N_STRIPE) // 8)
        def _():
            pltpu.sync_copy(slab.at[pl.ds(NS * _N_STRIPE + 8 * s, 8)],
                            out.at[c].at[pl.ds(NS * _N_STRIPE + 8 * s, 8)])

    return sc_attn


_sc_attn_l1 = _mk_sc_attn("l1")
_sc_attn_l2 = _mk_sc_attn("l2")


def kernel(x, edge_index, edge_attr, Wq, bq, Wk, bk, Wv, bv, Wskip, bskip,
           Wl, bl, Wr, br, att, bg, W1, b1, gamma, beta, W2, b2):
    edges = jnp.stack(
        [edge_index[0].reshape(ROWS, CH), edge_index[1].reshape(ROWS, CH)],
        axis=1)

    wcat = jnp.concatenate([Wq, Wk, Wv, Wskip], axis=1)
    bcat = jnp.concatenate([bq, bk, bv, bskip], axis=0)
    qkv, skip = _proj(x, wcat, bcat, bm=1000)
    tbl1 = qkv.reshape(3 * N, H)

    slabs1 = _sc_attn_l1(edges, tbl1, att)
    acc1 = slabs1[0] + slabs1[1]
    den1 = acc1[N:].reshape(-1)[0:N].reshape(N, 1)

    wlr = jnp.concatenate([Wl, Wr], axis=1)
    blr = jnp.concatenate([bl, br], axis=0)
    xlr, h = _norm_proj(acc1[0:N], den1, skip, wlr, blr, bm=1000)
    xl = xlr[0]
    xr = xlr[1]
    tbl2 = jnp.concatenate(
        [xlr.reshape(2 * N, H), jnp.zeros((N, H), jnp.float32)], axis=0)

    slabs2 = _sc_attn_l2(edges, tbl2, att)
    acc2 = slabs2[0] + slabs2[1]
    den2 = acc2[N:].reshape(-1)[0:N].reshape(N, 1)

    z = _finalize(acc2[0:N], den2, xl, xr, att, bg, bm=1000)

    ef = _edge_mlp(edge_attr, W1, b1, gamma, beta, W2, b2, bm=2000)
    return (z, ef)


# speedup vs baseline: 1.0846x; 1.0846x over previous
import functools
import jax
import jax.numpy as jnp
from jax import lax
from jax.experimental import pallas as pl
from jax.experimental.pallas import tpu as pltpu
from jax.experimental.pallas import tpu_sc as plsc

N = 10000
E = 320000
DIN = 128
H = 128
DE = 16

NC = 2
NS = 16
NW = NC * NS
CH = 80
ROWS = E // CH
_RSQRT_H = 1.0 / (128.0 ** 0.5)


def _proj_body(x_ref, w_ref, b_ref, t_ref, skip_ref):
    p = (
        jnp.dot(x_ref[...], w_ref[...], preferred_element_type=jnp.float32)
        + b_ref[...]
    )
    t_ref[0] = p[:, 0:H]
    t_ref[1] = p[:, H:2 * H]
    t_ref[2] = p[:, 2 * H:3 * H]
    skip_ref[...] = p[:, 3 * H:4 * H]


def _proj(x, w, b, bm):
    n = x.shape[0]
    return pl.pallas_call(
        _proj_body,
        grid=(n // bm,),
        in_specs=[
            pl.BlockSpec((bm, DIN), lambda i: (i, 0)),
            pl.BlockSpec((DIN, 4 * H), lambda i: (0, 0)),
            pl.BlockSpec((1, 4 * H), lambda i: (0, 0)),
        ],
        out_specs=[
            pl.BlockSpec((3, bm, H), lambda i: (0, i, 0)),
            pl.BlockSpec((bm, H), lambda i: (i, 0)),
        ],
        out_shape=[
            jax.ShapeDtypeStruct((3, n, H), jnp.float32),
            jax.ShapeDtypeStruct((n, H), jnp.float32),
        ],
    )(x, w, b.reshape(1, 4 * H))


def _norm_proj_body(a_ref, den_ref, skip_ref, w_ref, b_ref, o_ref, h_ref):
    num = a_ref[...]
    den = den_ref[...]
    h = jnp.maximum(num / (den + 1e-16) + skip_ref[...], 0.0)
    h_ref[...] = h
    o = jnp.dot(h, w_ref[...], preferred_element_type=jnp.float32) + b_ref[...]
    o_ref[0] = o[:, 0:H]
    o_ref[1] = o[:, H:2 * H]


def _norm_proj(slabs, den, skip, w, b, bm):
    n = skip.shape[0]
    dout = w.shape[1]
    return pl.pallas_call(
        _norm_proj_body,
        grid=(n // bm,),
        in_specs=[
            pl.BlockSpec((bm, H), lambda i: (i, 0)),
            pl.BlockSpec((bm, 1), lambda i: (i, 0)),
            pl.BlockSpec((bm, H), lambda i: (i, 0)),
            pl.BlockSpec((H, dout), lambda i: (0, 0)),
            pl.BlockSpec((1, dout), lambda i: (0, 0)),
        ],
        out_specs=[
            pl.BlockSpec((2, bm, H), lambda i: (0, i, 0)),
            pl.BlockSpec((bm, H), lambda i: (i, 0)),
        ],
        out_shape=[
            jax.ShapeDtypeStruct((2, n, H), jnp.float32),
            jax.ShapeDtypeStruct((n, H), jnp.float32),
        ],
    )(slabs, den, skip, w, b.reshape(1, dout))


def _fin_body(s_ref, den_ref, xl_ref, xr_ref, att_ref, bg_ref, o_ref):
    xl = xl_ref[...]
    s = xl + xr_ref[...]
    s = jnp.where(s > 0, s, 0.2 * s)
    eself = jnp.exp(jnp.sum(s * att_ref[...], axis=-1, keepdims=True))
    num = s_ref[...] + eself * xl
    den = den_ref[...] + eself + 1e-16
    o_ref[...] = num / den + bg_ref[...]


def _finalize(slabs, den, xl, xr, att, bg, bm):
    n = xl.shape[0]
    return pl.pallas_call(
        _fin_body,
        grid=(n // bm,),
        in_specs=[
            pl.BlockSpec((bm, H), lambda i: (i, 0)),
            pl.BlockSpec((bm, 1), lambda i: (i, 0)),
            pl.BlockSpec((bm, H), lambda i: (i, 0)),
            pl.BlockSpec((bm, H), lambda i: (i, 0)),
            pl.BlockSpec((1, H), lambda i: (0, 0)),
            pl.BlockSpec((1, H), lambda i: (0, 0)),
        ],
        out_specs=pl.BlockSpec((bm, H), lambda i: (i, 0)),
        out_shape=jax.ShapeDtypeStruct((n, H), jnp.float32),
    )(slabs, den, xl, xr, att.reshape(1, H), bg.reshape(1, H))


def _mlp_body(ea_ref, w1_ref, b1_ref, g_ref, be_ref, w2_ref, b2_ref, o_ref):
    he = (
        jnp.dot(ea_ref[...], w1_ref[...], preferred_element_type=jnp.float32)
        + b1_ref[...]
    )
    mu = jnp.mean(he, axis=-1, keepdims=True)
    var = jnp.mean((he - mu) ** 2, axis=-1, keepdims=True)
    he = (he - mu) * jax.lax.rsqrt(var + 1e-5) * g_ref[...] + be_ref[...]
    he = jnp.maximum(he, 0.0)
    o_ref[...] = (
        jnp.dot(he, w2_ref[...], preferred_element_type=jnp.float32)
        + b2_ref[...]
    )


def _edge_mlp(ea, w1, b1, gamma, beta, w2, b2, bm):
    e = ea.shape[0]
    return pl.pallas_call(
        _mlp_body,
        grid=(e // bm,),
        in_specs=[
            pl.BlockSpec((bm, DE), lambda i: (i, 0)),
            pl.BlockSpec((DE, H), lambda i: (0, 0)),
            pl.BlockSpec((1, H), lambda i: (0, 0)),
            pl.BlockSpec((1, H), lambda i: (0, 0)),
            pl.BlockSpec((1, H), lambda i: (0, 0)),
            pl.BlockSpec((H, H), lambda i: (0, 0)),
            pl.BlockSpec((1, H), lambda i: (0, 0)),
        ],
        out_specs=pl.BlockSpec((bm, H), lambda i: (i, 0)),
        out_shape=jax.ShapeDtypeStruct((e, H), jnp.float32),
    )(ea, w1, b1.reshape(1, H), gamma.reshape(1, H), beta.reshape(1, H),
      w2, b2.reshape(1, H))


_N_STRIPE = 624
_N_REM = N - NS * _N_STRIPE
DR = 80
N2 = N + DR


def _mk_sc_attn(mode):
    mesh = plsc.VectorSubcoreMesh(
        core_axis_name="c", subcore_axis_name="s", num_cores=NC,
        num_subcores=NS,
    )

    @functools.partial(
        pl.kernel,
        out_type=jax.ShapeDtypeStruct((NC, N2, H), jnp.float32),
        mesh=mesh,
        scratch_types=[
            pltpu.VMEM((2, CH), jnp.int32),
            pltpu.VMEM((CH,), jnp.int32),
            pltpu.VMEM((CH,), jnp.int32),
            pltpu.VMEM((CH,), jnp.int32),
            pltpu.VMEM((2 * CH,), jnp.int32),
            pltpu.VMEM((CH, H), jnp.float32),
            pltpu.VMEM((CH, H), jnp.float32),
            pltpu.VMEM((2 * CH, H), jnp.float32),
            pltpu.VMEM((H,), jnp.float32),
            pltpu.VMEM_SHARED((N2, H), jnp.float32),
            pltpu.SemaphoreType.DMA,
            pltpu.SemaphoreType.DMA,
            pltpu.SemaphoreType.DMA,
        ],
    )
    def sc_attn(edges, tbl, attv, out,
                idxb, aidx, vidx, bidx, scidx, abuf, bbuf, obuf, attbuf,
                slab, sem_a, sem_b, sem_v):
        c = lax.axis_index("c")
        s = lax.axis_index("s")
        w = s * NC + c

        lanes = lax.iota(jnp.int32, 16)
        ng = CH // 16

        def zrow(i, _):
            for j in range(8):
                obuf[i, pl.ds(16 * j, 16)] = jnp.zeros((16,), jnp.float32)
            return 0
        lax.fori_loop(0, 2 * CH, zrow, 0)

        for t in range(_N_STRIPE // (2 * CH)):
            pltpu.sync_copy(
                obuf, slab.at[pl.ds(s * _N_STRIPE + t * 2 * CH, 2 * CH)])
        _rem = _N_STRIPE % (2 * CH)
        if _rem:
            pltpu.sync_copy(
                obuf.at[pl.ds(0, _rem)],
                slab.at[pl.ds(s * _N_STRIPE + _N_STRIPE - _rem, _rem)])

        @pl.when(s < (N2 - NS * _N_STRIPE) // 8)
        def _():
            pltpu.sync_copy(obuf.at[pl.ds(0, 8)],
                            slab.at[pl.ds(NS * _N_STRIPE + 8 * s, 8)])

        pltpu.sync_copy(attv, attbuf)
        plsc.subcore_barrier()

        attregs = [attbuf[pl.ds(16 * j, 16)] for j in range(8)]

        _gdn = lax.GatherDimensionNumbers(
            offset_dims=(), collapsed_slice_dims=(0,), start_index_map=(0,))

        def _shuf(vec, idx):
            return lax.gather(
                vec, idx[:, None], _gdn, (1,),
                mode=lax.GatherScatterMode.PROMISE_IN_BOUNDS)

        def _allsum(vec):
            for kk in (8, 4, 2, 1):
                vec = vec + _shuf(vec, lanes ^ kk)
            return vec

        base = w * (ROWS // NW)

        def do_row(i, _):
            r = base + i
            pltpu.sync_copy(edges.at[r], idxb)
            for g in range(ng):
                sv = idxb[0, pl.ds(16 * g, 16)]
                dv = idxb[1, pl.ds(16 * g, 16)]
                if mode == "l1":
                    aidx[pl.ds(16 * g, 16)] = sv + N
                    vidx[pl.ds(16 * g, 16)] = sv + 2 * N
                    bidx[pl.ds(16 * g, 16)] = dv
                else:
                    aidx[pl.ds(16 * g, 16)] = sv
                    bidx[pl.ds(16 * g, 16)] = dv + N
                scidx[pl.ds(16 * g, 16)] = dv
                scidx[pl.ds(CH + 16 * g, 16)] = (
                    lax.shift_right_logical(dv, 7) + N)
            da = pltpu.async_copy(tbl.at[aidx], abuf, sem_a)
            db = pltpu.async_copy(tbl.at[bidx], bbuf, sem_b)
            if mode == "l1":
                dv_ = pltpu.async_copy(tbl.at[vidx],
                                       obuf.at[pl.ds(0, CH)], sem_v)
            da.wait()
            db.wait()
            if mode == "l1":
                dv_.wait()

            @plsc.parallel_loop(0, CH, unroll=2)
            def _edge(e):
                if mode == "l1":
                    acc = jnp.zeros((16,), jnp.float32)
                    for j in range(8):
                        a = abuf[e, pl.ds(16 * j, 16)]
                        b = bbuf[e, pl.ds(16 * j, 16)]
                        acc = acc + a * b
                    ex = jnp.exp(_allsum(acc) * _RSQRT_H)
                    for j in range(8):
                        v = obuf[e, pl.ds(16 * j, 16)]
                        obuf[e, pl.ds(16 * j, 16)] = v * ex
                else:
                    acc = jnp.zeros((16,), jnp.float32)
                    pay = []
                    for j in range(8):
                        a = abuf[e, pl.ds(16 * j, 16)]
                        pay.append(a)
                        t = a + bbuf[e, pl.ds(16 * j, 16)]
                        t = jnp.where(t > 0, t, 0.2 * t)
                        acc = acc + t * attregs[j]
                    ex = jnp.exp(_allsum(acc))
                    for j in range(8):
                        obuf[e, pl.ds(16 * j, 16)] = pay[j] * ex
                dvec = idxb[1, pl.ds((e // 16) * 16, 16)]
                dlo = _shuf(dvec, jnp.full((16,), e % 16, jnp.int32)) & 127
                for j in range(8):
                    obuf[CH + e, pl.ds(16 * j, 16)] = jnp.where(
                        lanes + 16 * j == dlo, ex, 0.0)

            pltpu.sync_copy(obuf, slab.at[scidx], add=True)
            return 0

        lax.fori_loop(0, ROWS // NW, do_row, 0)

        plsc.subcore_barrier()
        pltpu.sync_copy(slab.at[pl.ds(s * _N_STRIPE, _N_STRIPE)],
                        out.at[c].at[pl.ds(s * _N_STRIPE, _N_STRIPE)])

        @pl.when(s < (N2 - NS * _N_STRIPE) // 8)
        def _():
            pltpu.sync_copy(slab.at[pl.ds(NS * _N_STRIPE + 8 * s, 8)],
                            out.at[c].at[pl.ds(NS * _N_STRIPE + 8 * s, 8)])

    return sc_attn


_sc_attn_l1 = _mk_sc_attn("l1")
_sc_attn_l2 = _mk_sc_attn("l2")


def kernel(x, edge_index, edge_attr, Wq, bq, Wk, bk, Wv, bv, Wskip, bskip,
           Wl, bl, Wr, br, att, bg, W1, b1, gamma, beta, W2, b2):
    edges = jnp.stack(
        [edge_index[0].reshape(ROWS, CH), edge_index[1].reshape(ROWS, CH)],
        axis=1)

    wcat = jnp.concatenate([Wq, Wk, Wv, Wskip], axis=1)
    bcat = jnp.concatenate([bq, bk, bv, bskip], axis=0)
    qkv, skip = _proj(x, wcat, bcat, bm=1000)
    tbl1 = qkv.reshape(3 * N, H)

    slabs1 = _sc_attn_l1(edges, tbl1, att)
    acc1 = slabs1[0] + slabs1[1]
    den1 = acc1[N:].reshape(-1)[0:N].reshape(N, 1)

    wlr = jnp.concatenate([Wl, Wr], axis=1)
    blr = jnp.concatenate([bl, br], axis=0)
    xlr, h = _norm_proj(acc1[0:N], den1, skip, wlr, blr, bm=1000)
    xl = xlr[0]
    xr = xlr[1]
    tbl2 = jnp.concatenate(
        [xlr.reshape(2 * N, H), jnp.zeros((N, H), jnp.float32)], axis=0)

    slabs2 = _sc_attn_l2(edges, tbl2, att)
    acc2 = slabs2[0] + slabs2[1]
    den2 = acc2[N:].reshape(-1)[0:N].reshape(N, 1)

    z = _finalize(acc2[0:N], den2, xl, xr, att, bg, bm=1000)

    ef = _edge_mlp(edge_attr, W1, b1, gamma, beta, W2, b2, bm=2000)
    return (z, ef)


# speedup vs baseline: 1.1900x; 1.0971x over previous
import functools
import jax
import jax.numpy as jnp
from jax import lax
from jax.experimental import pallas as pl
from jax.experimental.pallas import tpu as pltpu
from jax.experimental.pallas import tpu_sc as plsc

N = 10000
E = 320000
DIN = 128
H = 128
DE = 16

NC = 2
NS = 16
NW = NC * NS
CH = 80
SB = 25
ROWS = E // CH
_RSQRT_H = 1.0 / (128.0 ** 0.5)


def _proj_body(x_ref, w_ref, b_ref, t_ref, skip_ref):
    p = (
        jnp.dot(x_ref[...], w_ref[...], preferred_element_type=jnp.float32)
        + b_ref[...]
    )
    t_ref[0] = p[:, 0:H]
    t_ref[1] = p[:, H:2 * H]
    t_ref[2] = p[:, 2 * H:3 * H]
    skip_ref[...] = p[:, 3 * H:4 * H]


def _proj(x, w, b, bm):
    n = x.shape[0]
    return pl.pallas_call(
        _proj_body,
        grid=(n // bm,),
        in_specs=[
            pl.BlockSpec((bm, DIN), lambda i: (i, 0)),
            pl.BlockSpec((DIN, 4 * H), lambda i: (0, 0)),
            pl.BlockSpec((1, 4 * H), lambda i: (0, 0)),
        ],
        out_specs=[
            pl.BlockSpec((3, bm, H), lambda i: (0, i, 0)),
            pl.BlockSpec((bm, H), lambda i: (i, 0)),
        ],
        out_shape=[
            jax.ShapeDtypeStruct((3, n, H), jnp.float32),
            jax.ShapeDtypeStruct((n, H), jnp.float32),
        ],
    )(x, w, b.reshape(1, 4 * H))


def _norm_proj_body(a_ref, den_ref, skip_ref, w_ref, b_ref, o_ref, h_ref):
    num = a_ref[...]
    den = den_ref[...]
    h = jnp.maximum(num / (den + 1e-16) + skip_ref[...], 0.0)
    h_ref[...] = h
    o = jnp.dot(h, w_ref[...], preferred_element_type=jnp.float32) + b_ref[...]
    o_ref[0] = o[:, 0:H]
    o_ref[1] = o[:, H:2 * H]


def _norm_proj(slabs, den, skip, w, b, bm):
    n = skip.shape[0]
    dout = w.shape[1]
    return pl.pallas_call(
        _norm_proj_body,
        grid=(n // bm,),
        in_specs=[
            pl.BlockSpec((bm, H), lambda i: (i, 0)),
            pl.BlockSpec((bm, 1), lambda i: (i, 0)),
            pl.BlockSpec((bm, H), lambda i: (i, 0)),
            pl.BlockSpec((H, dout), lambda i: (0, 0)),
            pl.BlockSpec((1, dout), lambda i: (0, 0)),
        ],
        out_specs=[
            pl.BlockSpec((2, bm, H), lambda i: (0, i, 0)),
            pl.BlockSpec((bm, H), lambda i: (i, 0)),
        ],
        out_shape=[
            jax.ShapeDtypeStruct((2, n, H), jnp.float32),
            jax.ShapeDtypeStruct((n, H), jnp.float32),
        ],
    )(slabs, den, skip, w, b.reshape(1, dout))


def _fin_body(s_ref, den_ref, xl_ref, xr_ref, att_ref, bg_ref, o_ref):
    xl = xl_ref[...]
    s = xl + xr_ref[...]
    s = jnp.where(s > 0, s, 0.2 * s)
    eself = jnp.exp(jnp.sum(s * att_ref[...], axis=-1, keepdims=True))
    num = s_ref[...] + eself * xl
    den = den_ref[...] + eself + 1e-16
    o_ref[...] = num / den + bg_ref[...]


def _finalize(slabs, den, xl, xr, att, bg, bm):
    n = xl.shape[0]
    return pl.pallas_call(
        _fin_body,
        grid=(n // bm,),
        in_specs=[
            pl.BlockSpec((bm, H), lambda i: (i, 0)),
            pl.BlockSpec((bm, 1), lambda i: (i, 0)),
            pl.BlockSpec((bm, H), lambda i: (i, 0)),
            pl.BlockSpec((bm, H), lambda i: (i, 0)),
            pl.BlockSpec((1, H), lambda i: (0, 0)),
            pl.BlockSpec((1, H), lambda i: (0, 0)),
        ],
        out_specs=pl.BlockSpec((bm, H), lambda i: (i, 0)),
        out_shape=jax.ShapeDtypeStruct((n, H), jnp.float32),
    )(slabs, den, xl, xr, att.reshape(1, H), bg.reshape(1, H))


def _mlp_body(ea_ref, w1_ref, b1_ref, g_ref, be_ref, w2_ref, b2_ref, o_ref):
    he = (
        jnp.dot(ea_ref[...], w1_ref[...], preferred_element_type=jnp.float32)
        + b1_ref[...]
    )
    mu = jnp.mean(he, axis=-1, keepdims=True)
    var = jnp.mean((he - mu) ** 2, axis=-1, keepdims=True)
    he = (he - mu) * jax.lax.rsqrt(var + 1e-5) * g_ref[...] + be_ref[...]
    he = jnp.maximum(he, 0.0)
    o_ref[...] = (
        jnp.dot(he, w2_ref[...], preferred_element_type=jnp.float32)
        + b2_ref[...]
    )


def _edge_mlp(ea, w1, b1, gamma, beta, w2, b2, bm):
    e = ea.shape[0]
    return pl.pallas_call(
        _mlp_body,
        grid=(e // bm,),
        in_specs=[
            pl.BlockSpec((bm, DE), lambda i: (i, 0)),
            pl.BlockSpec((DE, H), lambda i: (0, 0)),
            pl.BlockSpec((1, H), lambda i: (0, 0)),
            pl.BlockSpec((1, H), lambda i: (0, 0)),
            pl.BlockSpec((1, H), lambda i: (0, 0)),
            pl.BlockSpec((H, H), lambda i: (0, 0)),
            pl.BlockSpec((1, H), lambda i: (0, 0)),
        ],
        out_specs=pl.BlockSpec((bm, H), lambda i: (i, 0)),
        out_shape=jax.ShapeDtypeStruct((e, H), jnp.float32),
    )(ea, w1, b1.reshape(1, H), gamma.reshape(1, H), beta.reshape(1, H),
      w2, b2.reshape(1, H))


_N_STRIPE = 624
_N_REM = N - NS * _N_STRIPE
DR = 80
N2 = N + DR


def _mk_sc_attn(mode):
    mesh = plsc.VectorSubcoreMesh(
        core_axis_name="c", subcore_axis_name="s", num_cores=NC,
        num_subcores=NS,
    )

    @functools.partial(
        pl.kernel,
        out_type=jax.ShapeDtypeStruct((NC, N2, H), jnp.float32),
        mesh=mesh,
        scratch_types=[
            pltpu.VMEM((SB, 2, CH), jnp.int32),
            pltpu.VMEM((CH,), jnp.int32),
            pltpu.VMEM((CH,), jnp.int32),
            pltpu.VMEM((CH,), jnp.int32),
            pltpu.VMEM((2 * CH,), jnp.int32),
            pltpu.VMEM((CH, H), jnp.float32),
            pltpu.VMEM((CH, H), jnp.float32),
            pltpu.VMEM((2 * CH, H), jnp.float32),
            pltpu.VMEM((H,), jnp.float32),
            pltpu.VMEM_SHARED((N2, H), jnp.float32),
            pltpu.SemaphoreType.DMA,
            pltpu.SemaphoreType.DMA,
            pltpu.SemaphoreType.DMA,
        ],
    )
    def sc_attn(edges, tbl, attv, out,
                idxb, aidx, vidx, bidx, scidx, abuf, bbuf, obuf, attbuf,
                slab, sem_a, sem_b, sem_v):
        c = lax.axis_index("c")
        s = lax.axis_index("s")
        w = s * NC + c

        lanes = lax.iota(jnp.int32, 16)
        ng = CH // 16

        def zrow(i, _):
            for j in range(8):
                obuf[i, pl.ds(16 * j, 16)] = jnp.zeros((16,), jnp.float32)
            return 0
        lax.fori_loop(0, 2 * CH, zrow, 0)

        for t in range(_N_STRIPE // (2 * CH)):
            pltpu.sync_copy(
                obuf, slab.at[pl.ds(s * _N_STRIPE + t * 2 * CH, 2 * CH)])
        _rem = _N_STRIPE % (2 * CH)
        if _rem:
            pltpu.sync_copy(
                obuf.at[pl.ds(0, _rem)],
                slab.at[pl.ds(s * _N_STRIPE + _N_STRIPE - _rem, _rem)])

        @pl.when(s < (N2 - NS * _N_STRIPE) // 8)
        def _():
            pltpu.sync_copy(obuf.at[pl.ds(0, 8)],
                            slab.at[pl.ds(NS * _N_STRIPE + 8 * s, 8)])

        pltpu.sync_copy(attv, attbuf)
        plsc.subcore_barrier()

        attregs = [attbuf[pl.ds(16 * j, 16)] for j in range(8)]

        _gdn = lax.GatherDimensionNumbers(
            offset_dims=(), collapsed_slice_dims=(0,), start_index_map=(0,))

        def _shuf(vec, idx):
            return lax.gather(
                vec, idx[:, None], _gdn, (1,),
                mode=lax.GatherScatterMode.PROMISE_IN_BOUNDS)

        def _allsum(vec):
            for kk in (8, 4, 2, 1):
                vec = vec + _shuf(vec, lanes ^ kk)
            return vec

        nch = ROWS // NW
        base = w * nch

        def do_sb(t, _):
            pltpu.sync_copy(edges.at[pl.ds(base + SB * t, SB)], idxb)
            lax.fori_loop(0, SB, do_row, 0)
            return 0

        def do_row(i, _):
            for g in range(ng):
                sv = idxb[i, 0, pl.ds(16 * g, 16)]
                dv = idxb[i, 1, pl.ds(16 * g, 16)]
                if mode == "l1":
                    aidx[pl.ds(16 * g, 16)] = sv + N
                    vidx[pl.ds(16 * g, 16)] = sv + 2 * N
                    bidx[pl.ds(16 * g, 16)] = dv
                else:
                    aidx[pl.ds(16 * g, 16)] = sv
                    bidx[pl.ds(16 * g, 16)] = dv + N
                scidx[pl.ds(16 * g, 16)] = dv
                scidx[pl.ds(CH + 16 * g, 16)] = (
                    lax.shift_right_logical(dv, 7) + N)
            da = pltpu.async_copy(tbl.at[aidx], abuf, sem_a)
            db = pltpu.async_copy(tbl.at[bidx], bbuf, sem_b)
            if mode == "l1":
                dv_ = pltpu.async_copy(tbl.at[vidx],
                                       obuf.at[pl.ds(0, CH)], sem_v)
            da.wait()
            db.wait()
            if mode == "l1":
                dv_.wait()

            @plsc.parallel_loop(0, CH, unroll=2)
            def _edge(e):
                if mode == "l1":
                    acc = jnp.zeros((16,), jnp.float32)
                    for j in range(8):
                        a = abuf[e, pl.ds(16 * j, 16)]
                        b = bbuf[e, pl.ds(16 * j, 16)]
                        acc = acc + a * b
                    ex = jnp.exp(_allsum(acc) * _RSQRT_H)
                    for j in range(8):
                        v = obuf[e, pl.ds(16 * j, 16)]
                        obuf[e, pl.ds(16 * j, 16)] = v * ex
                else:
                    acc = jnp.zeros((16,), jnp.float32)
                    pay = []
                    for j in range(8):
                        a = abuf[e, pl.ds(16 * j, 16)]
                        pay.append(a)
                        t = a + bbuf[e, pl.ds(16 * j, 16)]
                        t = jnp.where(t > 0, t, 0.2 * t)
                        acc = acc + t * attregs[j]
                    ex = jnp.exp(_allsum(acc))
                    for j in range(8):
                        obuf[e, pl.ds(16 * j, 16)] = pay[j] * ex
                dvec = idxb[i, 1, pl.ds((e // 16) * 16, 16)]
                dlo = _shuf(dvec, jnp.full((16,), e % 16, jnp.int32)) & 127
                for j in range(8):
                    obuf[CH + e, pl.ds(16 * j, 16)] = jnp.where(
                        lanes + 16 * j == dlo, ex, 0.0)

            pltpu.sync_copy(obuf, slab.at[scidx], add=True)
            return 0

        lax.fori_loop(0, nch // SB, do_sb, 0)

        plsc.subcore_barrier()
        pltpu.sync_copy(slab.at[pl.ds(s * _N_STRIPE, _N_STRIPE)],
                        out.at[c].at[pl.ds(s * _N_STRIPE, _N_STRIPE)])

        @pl.when(s < (N2 - NS * _N_STRIPE) // 8)
        def _():
            pltpu.sync_copy(slab.at[pl.ds(NS * _N_STRIPE + 8 * s, 8)],
                            out.at[c].at[pl.ds(NS * _N_STRIPE + 8 * s, 8)])

    return sc_attn


_sc_attn_l1 = _mk_sc_attn("l1")
_sc_attn_l2 = _mk_sc_attn("l2")


def kernel(x, edge_index, edge_attr, Wq, bq, Wk, bk, Wv, bv, Wskip, bskip,
           Wl, bl, Wr, br, att, bg, W1, b1, gamma, beta, W2, b2):
    edges = jnp.stack(
        [edge_index[0].reshape(ROWS, CH), edge_index[1].reshape(ROWS, CH)],
        axis=1)

    wcat = jnp.concatenate([Wq, Wk, Wv, Wskip], axis=1)
    bcat = jnp.concatenate([bq, bk, bv, bskip], axis=0)
    qkv, skip = _proj(x, wcat, bcat, bm=1000)
    tbl1 = qkv.reshape(3 * N, H)

    slabs1 = _sc_attn_l1(edges, tbl1, att)
    acc1 = slabs1[0] + slabs1[1]
    den1 = acc1[N:].reshape(-1)[0:N].reshape(N, 1)

    wlr = jnp.concatenate([Wl, Wr], axis=1)
    blr = jnp.concatenate([bl, br], axis=0)
    xlr, h = _norm_proj(acc1[0:N], den1, skip, wlr, blr, bm=1000)
    xl = xlr[0]
    xr = xlr[1]
    tbl2 = jnp.concatenate(
        [xlr.reshape(2 * N, H), jnp.zeros((N, H), jnp.float32)], axis=0)

    slabs2 = _sc_attn_l2(edges, tbl2, att)
    acc2 = slabs2[0] + slabs2[1]
    den2 = acc2[N:].reshape(-1)[0:N].reshape(N, 1)

    z = _finalize(acc2[0:N], den2, xl, xr, att, bg, bm=1000)

    ef = _edge_mlp(edge_attr, W1, b1, gamma, beta, W2, b2, bm=2000)
    return (z, ef)
